# Initial kernel scaffold; baseline (speedup 1.0000x reference)
import jax
import jax.numpy as jnp
from jax.experimental import pallas as pl


def kernel(x, edge_index, W1, att_src1, att_dst1, b1, W2, att_src2, att_dst2, b2):
    raise NotImplementedError("write your pallas kernel here")


# speedup vs baseline: 27.8642x; 27.8642x over previous
import functools
import jax
import jax.numpy as jnp
from jax import lax
from jax.experimental import pallas as pl
from jax.experimental.pallas import tpu as pltpu, tpu_sc as plsc

NC = 2
NS = 16
L = 16
K = 1024
KC = K // 128


def _leaky(x):
    return jnp.where(x > 0, x, 0.2 * x)


def _tc1_body(x_ref, w_ref, as_ref, ad_ref, p_ref,
              ta0_ref, ta1_ref, td0_ref, td1_ref, i0_ref, i1_ref):
    h = jnp.dot(x_ref[...], w_ref[...], preferred_element_type=jnp.float32)
    asrc = jnp.dot(h, as_ref[...], preferred_element_type=jnp.float32)
    adst = jnp.dot(h, ad_ref[...], preferred_element_type=jnp.float32)
    ex = jnp.exp(_leaky(asrc + adst))
    ex_rep = jnp.dot(ex, p_ref[...], preferred_element_type=jnp.float32)
    nb = h.shape[0]
    z12 = jnp.zeros((nb, 12), jnp.float32)
    for g, (ta_ref, td_ref, init_ref) in enumerate(
            [(ta0_ref, td0_ref, i0_ref), (ta1_ref, td1_ref, i1_ref)]):
        hg = h[:, 32 * g:32 * g + 32]
        ta_ref[...] = jnp.concatenate(
            [hg, asrc[:, 4 * g:4 * g + 4], z12], axis=1)
        td_ref[...] = jnp.concatenate([adst[:, 4 * g:4 * g + 4], z12], axis=1)
        init_ref[...] = jnp.concatenate(
            [hg * ex_rep[:, 32 * g:32 * g + 32], ex[:, 4 * g:4 * g + 4], z12],
            axis=1)


def _tc_layer1(x, W1, As1, Ad1, P1, nblk):
    n, d = x.shape
    c1 = W1.shape[1]
    h1 = As1.shape[1]
    grid = n // nblk
    return pl.pallas_call(
        _tc1_body,
        grid=(grid,),
        in_specs=[
            pl.BlockSpec((nblk, d), lambda i: (i, 0)),
            pl.BlockSpec((d, c1), lambda i: (0, 0)),
            pl.BlockSpec((c1, h1), lambda i: (0, 0)),
            pl.BlockSpec((c1, h1), lambda i: (0, 0)),
            pl.BlockSpec((h1, c1), lambda i: (0, 0)),
        ],
        out_specs=[
            pl.BlockSpec((nblk, 48), lambda i: (i, 0)),
            pl.BlockSpec((nblk, 48), lambda i: (i, 0)),
            pl.BlockSpec((nblk, 16), lambda i: (i, 0)),
            pl.BlockSpec((nblk, 16), lambda i: (i, 0)),
            pl.BlockSpec((nblk, 48), lambda i: (i, 0)),
            pl.BlockSpec((nblk, 48), lambda i: (i, 0)),
        ],
        out_shape=[
            jax.ShapeDtypeStruct((n, 48), jnp.float32),
            jax.ShapeDtypeStruct((n, 48), jnp.float32),
            jax.ShapeDtypeStruct((n, 16), jnp.float32),
            jax.ShapeDtypeStruct((n, 16), jnp.float32),
            jax.ShapeDtypeStruct((n, 48), jnp.float32),
            jax.ShapeDtypeStruct((n, 48), jnp.float32),
        ],
    )(x, W1, As1, Ad1, P1)


def _tc2_body(a0_ref, a1_ref, p_ref, b1_ref, w2_ref, s2_ref, d2_ref,
              ts_ref, td_ref, init_ref):
    a0 = a0_ref[...]
    a1 = a1_ref[...]
    num = jnp.concatenate([a0[:, :32], a1[:, :32]], axis=1)
    den = jnp.concatenate([a0[:, 32:36], a1[:, 32:36]], axis=1)
    den_rep = jnp.dot(den, p_ref[...], preferred_element_type=jnp.float32)
    h1 = num / (den_rep + 1e-16) + b1_ref[...]
    h1 = jnp.where(h1 > 0, h1, jnp.exp(h1) - 1.0)
    h2 = jnp.dot(h1, w2_ref[...], preferred_element_type=jnp.float32)
    asrc = jnp.dot(h2, s2_ref[...], preferred_element_type=jnp.float32)
    adst = jnp.dot(h2, d2_ref[...], preferred_element_type=jnp.float32)
    ex = jnp.exp(_leaky(asrc + adst))
    nb = h2.shape[0]
    z8 = jnp.zeros((nb, 8), jnp.float32)
    ts_ref[...] = jnp.concatenate([h2, asrc, z8], axis=1)
    td_ref[...] = jnp.broadcast_to(adst, (nb, 16))
    init_ref[...] = jnp.concatenate([h2 * ex, ex, z8], axis=1)


def _tc_layer2(acc0, acc1, P1T, b1, W2, s2, d2, nblk):
    n = acc0.shape[0]
    c1 = W2.shape[0]
    c2 = W2.shape[1]
    grid = n // nblk
    return pl.pallas_call(
        _tc2_body,
        grid=(grid,),
        in_specs=[
            pl.BlockSpec((nblk, 48), lambda i: (i, 0)),
            pl.BlockSpec((nblk, 48), lambda i: (i, 0)),
            pl.BlockSpec((8, 64), lambda i: (0, 0)),
            pl.BlockSpec((1, c1), lambda i: (0, 0)),
            pl.BlockSpec((c1, c2), lambda i: (0, 0)),
            pl.BlockSpec((c2, 1), lambda i: (0, 0)),
            pl.BlockSpec((c2, 1), lambda i: (0, 0)),
        ],
        out_specs=[
            pl.BlockSpec((nblk, 16), lambda i: (i, 0)),
            pl.BlockSpec((nblk, 16), lambda i: (i, 0)),
            pl.BlockSpec((nblk, 16), lambda i: (i, 0)),
        ],
        out_shape=[
            jax.ShapeDtypeStruct((n, 16), jnp.float32),
            jax.ShapeDtypeStruct((n, 16), jnp.float32),
            jax.ShapeDtypeStruct((n, 16), jnp.float32),
        ],
    )(acc0, acc1, P1T, b1, W2, s2, d2)


def _tc3_body(acc_ref, b2_ref, out_ref):
    acc = acc_ref[...]
    nb = acc.shape[0]
    num = acc[:, :7]
    den = acc[:, 7:8]
    v = num / (den + 1e-16) + b2_ref[...]
    m = jnp.max(v, axis=1, keepdims=True)
    s = jnp.sum(jnp.exp(v - m), axis=1, keepdims=True)
    res = v - m - jnp.log(s)
    out_ref[...] = jnp.concatenate([res, jnp.zeros((nb, 9), jnp.float32)],
                                   axis=1)


def _tc_final(acc2, b2, nblk):
    n = acc2.shape[0]
    grid = n // nblk
    return pl.pallas_call(
        _tc3_body,
        grid=(grid,),
        in_specs=[
            pl.BlockSpec((nblk, 16), lambda i: (i, 0)),
            pl.BlockSpec((1, 7), lambda i: (0, 0)),
        ],
        out_specs=pl.BlockSpec((nblk, 16), lambda i: (i, 0)),
        out_shape=jax.ShapeDtypeStruct((n, 16), jnp.float32),
    )(acc2, b2)


def _make_sc_pass(n, e, epad, nh, nhpad, width, layer, k):
    kc = k // 128
    rounds = epad // (NS * k)
    rpt = nhpad // NS
    mesh = plsc.VectorSubcoreMesh(core_axis_name="c", subcore_axis_name="s",
                                  num_cores=NC, num_subcores=NS)

    @functools.partial(
        pl.kernel, mesh=mesh,
        compiler_params=pltpu.CompilerParams(needs_layout_passes=False,
                                             use_tc_tiling_on_sc=False),
        out_type=jax.ShapeDtypeStruct((NC * nhpad, width), jnp.float32),
        scratch_types=[
            pltpu.VMEM((k,), jnp.int32),
            pltpu.VMEM((k,), jnp.int32),
            pltpu.VMEM((kc, 128), jnp.int32),
            pltpu.VMEM((k, width), jnp.float32),
            pltpu.VMEM((k, 16), jnp.float32),
            pltpu.SemaphoreType.DMA,
            pltpu.SemaphoreType.DMA,
            pltpu.VMEM_SHARED((nhpad, width), jnp.float32),
        ],
    )
    def sc_pass(ta_hbm, td_hbm, init_hbm, src_hbm, dst_hbm, out_hbm,
                srcv, dstv, sidx, buf, bufd, sema, semb, accum):
        c = lax.axis_index("c")
        s = lax.axis_index("s")
        base_node = c * nh
        r0 = s * rpt
        pltpu.sync_copy(init_hbm.at[pl.ds(base_node + r0, rpt)],
                        accum.at[pl.ds(r0, rpt)])
        plsc.subcore_barrier()

        iota16 = lax.iota(jnp.int32, L)
        one = jnp.full((L,), 1.0, jnp.float32)
        zero = jnp.full((L,), 0.0, jnp.float32)
        mask4 = jnp.where(iota16 < 4, one, zero)
        maska = jnp.where(iota16 < 7, one, zero)
        maskb = jnp.where(iota16 == 7, one, zero)

        def round_body(r, _):
            e_base = (r * NS + s) * k
            pltpu.sync_copy(src_hbm.at[pl.ds(e_base, k)], srcv)
            pltpu.sync_copy(dst_hbm.at[pl.ds(e_base, k)], dstv)
            cps = []
            for j in range(kc):
                sl = pl.ds(j * 128, 128)
                cps.append(pltpu.async_copy(
                    ta_hbm.at[srcv.at[sl]], buf.at[sl], sema))
                cps.append(pltpu.async_copy(
                    td_hbm.at[dstv.at[sl]], bufd.at[sl], semb))
            for cp in cps:
                cp.wait()

            def sidx_body(jj, _):
                d16 = plsc.load_gather(dstv, [jj * L + iota16])
                ids = e_base + jj * L + iota16
                own = ((d16 >= base_node) & (d16 < base_node + nh)
                       & (ids < e))
                si = jnp.where(own, d16 - base_node, nh)
                rowv = jnp.full((L,), jj // 8, jnp.int32)
                colv = (jj % 8) * L + iota16
                plsc.store_scatter(sidx, [rowv, colv], si)
                return 0
            lax.fori_loop(0, k // L, sidx_body, 0)

            if layer == 1:
                c32 = 32 + iota16

                def row_body(rr, _):
                    rvec = jnp.full((L,), rr, jnp.int32)
                    va = (plsc.load_gather(buf, [rvec, c32])
                          + plsc.load_gather(bufd, [rvec, iota16]))
                    ex = jnp.exp(_leaky(va))
                    plsc.store_scatter(bufd, [rvec, iota16], ex)
                    for j in range(2):
                        colj = 2 * j + iota16 // 8
                        m = plsc.load_gather(bufd, [rvec, colj])
                        cj = j * L + iota16
                        v = plsc.load_gather(buf, [rvec, cj]) * m
                        plsc.store_scatter(buf, [rvec, cj], v)
                    plsc.store_scatter(buf, [rvec, c32], ex * mask4)
                    return 0
            else:
                c7 = jnp.full((L,), 7, jnp.int32)

                def row_body(rr, _):
                    rvec = jnp.full((L,), rr, jnp.int32)
                    vs = plsc.load_gather(buf, [rvec, iota16])
                    vd = plsc.load_gather(bufd, [rvec, iota16])
                    asr = plsc.load_gather(buf, [rvec, c7])
                    ex = jnp.exp(_leaky(asr + vd))
                    out = vs * ex * maska + ex * maskb
                    plsc.store_scatter(buf, [rvec, iota16], out)
                    return 0
            lax.fori_loop(0, k, row_body, 0)

            for j in range(kc):
                pltpu.sync_copy(buf.at[pl.ds(j * 128, 128)],
                                accum.at[sidx.at[j]], add=True)
            return 0
        lax.fori_loop(0, rounds, round_body, 0)
        plsc.subcore_barrier()
        pltpu.sync_copy(accum.at[pl.ds(r0, rpt)],
                        out_hbm.at[pl.ds(c * nhpad + r0, rpt)])

    return sc_pass


def kernel(x, edge_index, W1, att_src1, att_dst1, b1,
           W2, att_src2, att_dst2, b2):
    n, d = x.shape
    e = edge_index.shape[1]
    h1, f1 = att_src1.shape
    c1 = h1 * f1
    c2 = W2.shape[1]

    nh = (n + 1) // 2
    nhpad = ((nh + 1 + 127) // 128) * 128
    epad = ((e + NS * K - 1) // (NS * K)) * (NS * K)

    eye_h = jnp.eye(h1, dtype=jnp.float32)
    blk = jnp.repeat(eye_h, f1, axis=0)
    As1 = blk * att_src1.reshape(c1)[:, None]
    Ad1 = blk * att_dst1.reshape(c1)[:, None]
    P1 = blk.T
    s2 = att_src2.reshape(c2, 1)
    d2 = att_dst2.reshape(c2, 1)

    src = edge_index[0].astype(jnp.int32)
    dst = edge_index[1].astype(jnp.int32)
    src_p = jnp.pad(src, (0, epad - e))
    dst_p = jnp.pad(dst, (0, epad - e))

    ta_g0, ta_g1, td_g0, td_g1, in_g0, in_g1 = _tc_layer1(
        x, W1, As1, Ad1, P1, 400)
    pad_rows = ((0, nh + nhpad - n), (0, 0))
    sc1 = _make_sc_pass(n, e, epad, nh, nhpad, 48, 1, 512)
    acc_g0 = sc1(ta_g0, td_g0, jnp.pad(in_g0, pad_rows), src_p, dst_p)
    acc_g1 = sc1(ta_g1, td_g1, jnp.pad(in_g1, pad_rows), src_p, dst_p)

    def _stitch(a):
        return jnp.concatenate([a[:nh], a[nhpad:nhpad + n - nh]], axis=0)
    acc_g0 = _stitch(acc_g0)
    acc_g1 = _stitch(acc_g1)

    ta2, td2, init2 = _tc_layer2(acc_g0, acc_g1, P1, b1.reshape(1, c1),
                                 W2, s2, d2, 400)
    sc2 = _make_sc_pass(n, e, epad, nh, nhpad, 16, 2, 1024)
    acc2 = sc2(ta2, td2, jnp.pad(init2, pad_rows), src_p, dst_p)
    acc2 = _stitch(acc2)

    out = _tc_final(acc2, b2.reshape(1, c2), 400)
    return out[:, :c2]


# speedup vs baseline: 33.6898x; 1.2091x over previous
import functools
import jax
import jax.numpy as jnp
from jax import lax
from jax.experimental import pallas as pl
from jax.experimental.pallas import tpu as pltpu, tpu_sc as plsc

NC = 2
NS = 16
L = 16
K = 1024
KC = K // 128


def _leaky(x):
    return jnp.where(x > 0, x, 0.2 * x)


def _tc1_body(x_ref, w_ref, as_ref, ad_ref, p_ref,
              ta0_ref, ta1_ref, td0_ref, td1_ref, i0_ref, i1_ref):
    h = jnp.dot(x_ref[...], w_ref[...], preferred_element_type=jnp.float32)
    asrc = jnp.dot(h, as_ref[...], preferred_element_type=jnp.float32)
    adst = jnp.dot(h, ad_ref[...], preferred_element_type=jnp.float32)
    ex = jnp.exp(_leaky(asrc + adst))
    ex_rep = jnp.dot(ex, p_ref[...], preferred_element_type=jnp.float32)
    nb = h.shape[0]
    z12 = jnp.zeros((nb, 12), jnp.float32)
    for g, (ta_ref, td_ref, init_ref) in enumerate(
            [(ta0_ref, td0_ref, i0_ref), (ta1_ref, td1_ref, i1_ref)]):
        hg = h[:, 32 * g:32 * g + 32]
        ta_ref[...] = jnp.concatenate(
            [hg, asrc[:, 4 * g:4 * g + 4], z12], axis=1)
        td_ref[...] = jnp.concatenate([adst[:, 4 * g:4 * g + 4], z12], axis=1)
        init_ref[...] = jnp.concatenate(
            [hg * ex_rep[:, 32 * g:32 * g + 32], ex[:, 4 * g:4 * g + 4], z12],
            axis=1)


def _tc_layer1(x, W1, As1, Ad1, P1, nblk):
    n, d = x.shape
    c1 = W1.shape[1]
    h1 = As1.shape[1]
    grid = n // nblk
    return pl.pallas_call(
        _tc1_body,
        grid=(grid,),
        in_specs=[
            pl.BlockSpec((nblk, d), lambda i: (i, 0)),
            pl.BlockSpec((d, c1), lambda i: (0, 0)),
            pl.BlockSpec((c1, h1), lambda i: (0, 0)),
            pl.BlockSpec((c1, h1), lambda i: (0, 0)),
            pl.BlockSpec((h1, c1), lambda i: (0, 0)),
        ],
        out_specs=[
            pl.BlockSpec((nblk, 48), lambda i: (i, 0)),
            pl.BlockSpec((nblk, 48), lambda i: (i, 0)),
            pl.BlockSpec((nblk, 16), lambda i: (i, 0)),
            pl.BlockSpec((nblk, 16), lambda i: (i, 0)),
            pl.BlockSpec((nblk, 48), lambda i: (i, 0)),
            pl.BlockSpec((nblk, 48), lambda i: (i, 0)),
        ],
        out_shape=[
            jax.ShapeDtypeStruct((n, 48), jnp.float32),
            jax.ShapeDtypeStruct((n, 48), jnp.float32),
            jax.ShapeDtypeStruct((n, 16), jnp.float32),
            jax.ShapeDtypeStruct((n, 16), jnp.float32),
            jax.ShapeDtypeStruct((n, 48), jnp.float32),
            jax.ShapeDtypeStruct((n, 48), jnp.float32),
        ],
    )(x, W1, As1, Ad1, P1)


def _tc2_body(a0_ref, a1_ref, p_ref, b1_ref, w2_ref, s2_ref, d2_ref,
              ts_ref, td_ref, init_ref):
    a0 = a0_ref[...]
    a1 = a1_ref[...]
    num = jnp.concatenate([a0[:, :32], a1[:, :32]], axis=1)
    den = jnp.concatenate([a0[:, 32:36], a1[:, 32:36]], axis=1)
    den_rep = jnp.dot(den, p_ref[...], preferred_element_type=jnp.float32)
    h1 = num / (den_rep + 1e-16) + b1_ref[...]
    h1 = jnp.where(h1 > 0, h1, jnp.exp(h1) - 1.0)
    h2 = jnp.dot(h1, w2_ref[...], preferred_element_type=jnp.float32)
    asrc = jnp.dot(h2, s2_ref[...], preferred_element_type=jnp.float32)
    adst = jnp.dot(h2, d2_ref[...], preferred_element_type=jnp.float32)
    ex = jnp.exp(_leaky(asrc + adst))
    nb = h2.shape[0]
    z8 = jnp.zeros((nb, 8), jnp.float32)
    ts_ref[...] = jnp.concatenate([h2, asrc, z8], axis=1)
    td_ref[...] = jnp.broadcast_to(adst, (nb, 16))
    init_ref[...] = jnp.concatenate([h2 * ex, ex, z8], axis=1)


def _tc_layer2(acc0, acc1, P1T, b1, W2, s2, d2, nblk):
    n = acc0.shape[0]
    c1 = W2.shape[0]
    c2 = W2.shape[1]
    grid = n // nblk
    return pl.pallas_call(
        _tc2_body,
        grid=(grid,),
        in_specs=[
            pl.BlockSpec((nblk, 48), lambda i: (i, 0)),
            pl.BlockSpec((nblk, 48), lambda i: (i, 0)),
            pl.BlockSpec((8, 64), lambda i: (0, 0)),
            pl.BlockSpec((1, c1), lambda i: (0, 0)),
            pl.BlockSpec((c1, c2), lambda i: (0, 0)),
            pl.BlockSpec((c2, 1), lambda i: (0, 0)),
            pl.BlockSpec((c2, 1), lambda i: (0, 0)),
        ],
        out_specs=[
            pl.BlockSpec((nblk, 16), lambda i: (i, 0)),
            pl.BlockSpec((nblk, 16), lambda i: (i, 0)),
            pl.BlockSpec((nblk, 16), lambda i: (i, 0)),
        ],
        out_shape=[
            jax.ShapeDtypeStruct((n, 16), jnp.float32),
            jax.ShapeDtypeStruct((n, 16), jnp.float32),
            jax.ShapeDtypeStruct((n, 16), jnp.float32),
        ],
    )(acc0, acc1, P1T, b1, W2, s2, d2)


def _tc3_body(acc_ref, b2_ref, out_ref):
    acc = acc_ref[...]
    nb = acc.shape[0]
    num = acc[:, :7]
    den = acc[:, 7:8]
    v = num / (den + 1e-16) + b2_ref[...]
    m = jnp.max(v, axis=1, keepdims=True)
    s = jnp.sum(jnp.exp(v - m), axis=1, keepdims=True)
    res = v - m - jnp.log(s)
    out_ref[...] = jnp.concatenate([res, jnp.zeros((nb, 9), jnp.float32)],
                                   axis=1)


def _tc_final(acc2, b2, nblk):
    n = acc2.shape[0]
    grid = n // nblk
    return pl.pallas_call(
        _tc3_body,
        grid=(grid,),
        in_specs=[
            pl.BlockSpec((nblk, 16), lambda i: (i, 0)),
            pl.BlockSpec((1, 7), lambda i: (0, 0)),
        ],
        out_specs=pl.BlockSpec((nblk, 16), lambda i: (i, 0)),
        out_shape=jax.ShapeDtypeStruct((n, 16), jnp.float32),
    )(acc2, b2)


def _make_sc_pass(n, e, epad, nh, nhpad, width, layer, k):
    kc = k // 128
    rounds = epad // (NS * k)
    rpt = nhpad // NS
    mesh = plsc.VectorSubcoreMesh(core_axis_name="c", subcore_axis_name="s",
                                  num_cores=NC, num_subcores=NS)

    @functools.partial(
        pl.kernel, mesh=mesh,
        compiler_params=pltpu.CompilerParams(needs_layout_passes=False,
                                             use_tc_tiling_on_sc=False),
        out_type=jax.ShapeDtypeStruct((NC * nhpad, width), jnp.float32),
        scratch_types=[
            pltpu.VMEM((k,), jnp.int32),
            pltpu.VMEM((k,), jnp.int32),
            pltpu.VMEM((kc, 128), jnp.int32),
            pltpu.VMEM((k, width), jnp.float32),
            pltpu.VMEM((k, 16), jnp.float32),
            pltpu.SemaphoreType.DMA,
            pltpu.SemaphoreType.DMA,
            pltpu.VMEM_SHARED((nhpad, width), jnp.float32),
        ],
    )
    def sc_pass(ta_hbm, td_hbm, init_hbm, src_hbm, dst_hbm, out_hbm,
                srcv, dstv, sidx, buf, bufd, sema, semb, accum):
        c = lax.axis_index("c")
        s = lax.axis_index("s")
        base_node = c * nh
        r0 = s * rpt
        pltpu.sync_copy(init_hbm.at[pl.ds(base_node + r0, rpt)],
                        accum.at[pl.ds(r0, rpt)])
        plsc.subcore_barrier()

        iota16 = lax.iota(jnp.int32, L)
        one = jnp.full((L,), 1.0, jnp.float32)
        zero = jnp.full((L,), 0.0, jnp.float32)
        mask4 = jnp.where(iota16 < 4, one, zero)
        maska = jnp.where(iota16 < 7, one, zero)
        maskb = jnp.where(iota16 == 7, one, zero)

        def round_body(r, _):
            e_base = (r * NS + s) * k
            pltpu.sync_copy(src_hbm.at[pl.ds(e_base, k)], srcv)
            pltpu.sync_copy(dst_hbm.at[pl.ds(e_base, k)], dstv)
            cps = []
            for j in range(kc):
                sl = pl.ds(j * 128, 128)
                cps.append(pltpu.async_copy(
                    ta_hbm.at[srcv.at[sl]], buf.at[sl], sema))
                cps.append(pltpu.async_copy(
                    td_hbm.at[dstv.at[sl]], bufd.at[sl], semb))
            for cp in cps:
                cp.wait()

            def sidx_body(jj, _):
                d16 = plsc.load_gather(dstv, [jj * L + iota16])
                ids = e_base + jj * L + iota16
                own = ((d16 >= base_node) & (d16 < base_node + nh)
                       & (ids < e))
                si = jnp.where(own, d16 - base_node, nh)
                rowv = jnp.full((L,), jj // 8, jnp.int32)
                colv = (jj % 8) * L + iota16
                plsc.store_scatter(sidx, [rowv, colv], si)
                return 0
            lax.fori_loop(0, k // L, sidx_body, 0)

            if layer == 1:
                c32 = 32 + iota16
                q_rows = iota16 // 4
                q_cols = iota16 % 4

                def alpha_body(q, _):
                    rows = q * 4 + q_rows
                    va = (plsc.load_gather(buf, [rows, 32 + q_cols])
                          + plsc.load_gather(bufd, [rows, q_cols]))
                    ex = jnp.exp(_leaky(va))
                    plsc.store_scatter(bufd, [rows, q_cols], ex)
                    return 0
                lax.fori_loop(0, k // 4, alpha_body, 0)

                def row_body(rr, _):
                    rvec = jnp.full((L,), rr, jnp.int32)
                    for j in range(2):
                        colj = 2 * j + iota16 // 8
                        m = plsc.load_gather(bufd, [rvec, colj])
                        cj = j * L + iota16
                        v = plsc.load_gather(buf, [rvec, cj]) * m
                        plsc.store_scatter(buf, [rvec, cj], v)
                    den = plsc.load_gather(bufd, [rvec, q_cols]) * mask4
                    plsc.store_scatter(buf, [rvec, c32], den)
                    return 0
            else:
                c7 = jnp.full((L,), 7, jnp.int32)
                zero16 = jnp.full((L,), 0, jnp.int32)

                def alpha_body(q, _):
                    rows = q * L + iota16
                    va = (plsc.load_gather(buf, [rows, c7])
                          + plsc.load_gather(bufd, [rows, zero16]))
                    ex = jnp.exp(_leaky(va))
                    plsc.store_scatter(bufd, [rows, zero16], ex)
                    return 0
                lax.fori_loop(0, k // L, alpha_body, 0)

                def row_body(rr, _):
                    rvec = jnp.full((L,), rr, jnp.int32)
                    vs = plsc.load_gather(buf, [rvec, iota16])
                    m = plsc.load_gather(bufd, [rvec, zero16])
                    out = vs * m * maska + m * maskb
                    plsc.store_scatter(buf, [rvec, iota16], out)
                    return 0
            lax.fori_loop(0, k, row_body, 0)

            for j in range(kc):
                pltpu.sync_copy(buf.at[pl.ds(j * 128, 128)],
                                accum.at[sidx.at[j]], add=True)
            return 0
        lax.fori_loop(0, rounds, round_body, 0)
        plsc.subcore_barrier()
        pltpu.sync_copy(accum.at[pl.ds(r0, rpt)],
                        out_hbm.at[pl.ds(c * nhpad + r0, rpt)])

    return sc_pass


def kernel(x, edge_index, W1, att_src1, att_dst1, b1,
           W2, att_src2, att_dst2, b2):
    n, d = x.shape
    e = edge_index.shape[1]
    h1, f1 = att_src1.shape
    c1 = h1 * f1
    c2 = W2.shape[1]

    nh = (n + 1) // 2
    nhpad = ((nh + 1 + 127) // 128) * 128
    epad = ((e + NS * K - 1) // (NS * K)) * (NS * K)

    eye_h = jnp.eye(h1, dtype=jnp.float32)
    blk = jnp.repeat(eye_h, f1, axis=0)
    As1 = blk * att_src1.reshape(c1)[:, None]
    Ad1 = blk * att_dst1.reshape(c1)[:, None]
    P1 = blk.T
    s2 = att_src2.reshape(c2, 1)
    d2 = att_dst2.reshape(c2, 1)

    src = edge_index[0].astype(jnp.int32)
    dst = edge_index[1].astype(jnp.int32)
    src_p = jnp.pad(src, (0, epad - e))
    dst_p = jnp.pad(dst, (0, epad - e))

    ta_g0, ta_g1, td_g0, td_g1, in_g0, in_g1 = _tc_layer1(
        x, W1, As1, Ad1, P1, 400)
    pad_rows = ((0, nh + nhpad - n), (0, 0))
    sc1 = _make_sc_pass(n, e, epad, nh, nhpad, 48, 1, 512)
    acc_g0 = sc1(ta_g0, td_g0, jnp.pad(in_g0, pad_rows), src_p, dst_p)
    acc_g1 = sc1(ta_g1, td_g1, jnp.pad(in_g1, pad_rows), src_p, dst_p)

    def _stitch(a):
        return jnp.concatenate([a[:nh], a[nhpad:nhpad + n - nh]], axis=0)
    acc_g0 = _stitch(acc_g0)
    acc_g1 = _stitch(acc_g1)

    ta2, td2, init2 = _tc_layer2(acc_g0, acc_g1, P1, b1.reshape(1, c1),
                                 W2, s2, d2, 400)
    sc2 = _make_sc_pass(n, e, epad, nh, nhpad, 16, 2, 1024)
    acc2 = sc2(ta2, td2, jnp.pad(init2, pad_rows), src_p, dst_p)
    acc2 = _stitch(acc2)

    out = _tc_final(acc2, b2.reshape(1, c2), 400)
    return out[:, :c2]
